# Initial kernel scaffold; baseline (speedup 1.0000x reference)
import jax
import jax.numpy as jnp
from jax.experimental import pallas as pl


def kernel(x, he_index, he_attr, he_count, W1n, b1n, g1n, t1n, W2n, b2n, W1e, b1e, g1e, t1e, W2e, b2e, gf, bf):
    raise NotImplementedError("write your pallas kernel here")


# speedup vs baseline: 3.6909x; 3.6909x over previous
import functools

import jax
import jax.numpy as jnp
from jax import lax
from jax.experimental import pallas as pl
from jax.experimental.pallas import tpu as pltpu
from jax.experimental.pallas import tpu_sc as plsc

N = 10000
E = 10000
M = 320000
DN = 128
DHA = 16
HID = 128

NC = 2
NS = 16
NW = NC * NS
CH = 128
NCH = 80
MPAD = NW * NCH * CH
JUNK = 10000
ACC_R = 10240
PTR = ACC_R // NS
RB = 1000


def _ln(h, g, t, eps=1e-5):
    mu = jnp.mean(h, axis=-1, keepdims=True)
    var = jnp.mean((h - mu) ** 2, axis=-1, keepdims=True)
    return (h - mu) * lax.rsqrt(var + eps) * g + t


def _node_mlp_body(x_r, w1_r, b1_r, g1_r, t1_r, w2_r, b2_r, o_r):
    h = jnp.dot(x_r[...], w1_r[...], preferred_element_type=jnp.float32)
    h = _ln(h + b1_r[...], g1_r[...], t1_r[...])
    h = jnp.maximum(h, 0.0)
    o_r[...] = jnp.dot(h, w2_r[...], preferred_element_type=jnp.float32) + b2_r[...]


def _node_mlp(x, w1, b1, g1, t1, w2, b2):
    nb = N // RB
    full = lambda i: (0, 0)
    vec = lambda i: (0, 0)
    return pl.pallas_call(
        _node_mlp_body,
        grid=(nb,),
        in_specs=[
            pl.BlockSpec((RB, DN), lambda i: (i, 0)),
            pl.BlockSpec((DN, HID), full),
            pl.BlockSpec((1, HID), vec),
            pl.BlockSpec((1, HID), vec),
            pl.BlockSpec((1, HID), vec),
            pl.BlockSpec((HID, HID), full),
            pl.BlockSpec((1, HID), vec),
        ],
        out_specs=pl.BlockSpec((RB, HID), lambda i: (i, 0)),
        out_shape=jax.ShapeDtypeStruct((N, HID), jnp.float32),
    )(x, w1, b1, g1, t1, w2, b2)


def _edge_mlp_body(aggp_r, hc_r, attr_r, w1a_r, w1b_r, b1_r, g1_r, t1_r,
                   w2_r, b2_r, o_r):
    aggp = aggp_r[...]
    agg = (aggp[0] + aggp[1]) / (hc_r[...] + 1e-6)
    m = (jnp.dot(attr_r[...], w1a_r[...], preferred_element_type=jnp.float32)
         + jnp.dot(agg, w1b_r[...], preferred_element_type=jnp.float32))
    m = _ln(m + b1_r[...], g1_r[...], t1_r[...])
    m = jnp.maximum(m, 0.0)
    o = jnp.dot(m, w2_r[...], preferred_element_type=jnp.float32) + b2_r[...]
    o_r[...] = jnp.maximum(o, 0.0)


def _edge_mlp(agg_p, hc2, he_attr, w1a, w1b, b1, g1, t1, w2, b2):
    nb = E // RB
    full = lambda i: (0, 0)
    return pl.pallas_call(
        _edge_mlp_body,
        grid=(nb,),
        in_specs=[
            pl.BlockSpec((NC, RB, HID), lambda i: (0, i, 0)),
            pl.BlockSpec((RB, 1), lambda i: (i, 0)),
            pl.BlockSpec((RB, DHA), lambda i: (i, 0)),
            pl.BlockSpec((DHA, HID), full),
            pl.BlockSpec((HID, HID), full),
            pl.BlockSpec((1, HID), full),
            pl.BlockSpec((1, HID), full),
            pl.BlockSpec((1, HID), full),
            pl.BlockSpec((HID, DN), full),
            pl.BlockSpec((1, DN), full),
        ],
        out_specs=pl.BlockSpec((RB, DN), lambda i: (i, 0)),
        out_shape=jax.ShapeDtypeStruct((E, DN), jnp.float32),
    )(agg_p, hc2, he_attr, w1a, w1b, b1, g1, t1, w2, b2)


def _final_body(x_r, outp_r, degp_r, gf_r, bf_r, o_r):
    outp = outp_r[...]
    degp = degp_r[...]
    o = (outp[0] + outp[1]) / (degp[0] + degp[1] + 1e-6)
    o_r[...] = x_r[...] + _ln(o, gf_r[...], bf_r[...])


def _final(x, out_p, deg_p3, gf, bf):
    nb = N // RB
    return pl.pallas_call(
        _final_body,
        grid=(nb,),
        in_specs=[
            pl.BlockSpec((RB, DN), lambda i: (i, 0)),
            pl.BlockSpec((NC, RB, DN), lambda i: (0, i, 0)),
            pl.BlockSpec((NC, RB, 1), lambda i: (0, i, 0)),
            pl.BlockSpec((1, DN), lambda i: (0, 0)),
            pl.BlockSpec((1, DN), lambda i: (0, 0)),
        ],
        out_specs=pl.BlockSpec((RB, DN), lambda i: (i, 0)),
        out_shape=jax.ShapeDtypeStruct((N, DN), jnp.float32),
    )(x, out_p, deg_p3, gf, bf)


@functools.lru_cache(maxsize=None)
def _build_seg_sum():
    mesh = plsc.VectorSubcoreMesh(core_axis_name="c", subcore_axis_name="s",
                                  num_cores=NC, num_subcores=NS)

    @functools.partial(
        pl.kernel,
        out_type=[jax.ShapeDtypeStruct((NC, ACC_R, HID), jnp.float32)],
        mesh=mesh,
        scratch_types=[
            pltpu.VMEM((NCH, CH), jnp.int32),
            pltpu.VMEM((NCH, CH), jnp.int32),
            pltpu.VMEM((CH, HID), jnp.float32),
            pltpu.SemaphoreType.DMA,
            pltpu.VMEM_SHARED((ACC_R, HID), jnp.float32),
        ],
    )
    def seg_sum(table, srci, dsti, z2d, out_acc, srcv, dstv, rows, sem, acc_sh):
        c = lax.axis_index("c")
        s = lax.axis_index("s")
        wid = c * NS + s
        r0 = s * PTR
        pltpu.sync_copy(z2d, acc_sh.at[pl.ds(r0, PTR)])
        pltpu.sync_copy(srci.at[wid], srcv)
        pltpu.sync_copy(dsti.at[wid], dstv)
        plsc.subcore_barrier()

        def body(j, carry):
            pltpu.async_copy(table.at[srcv.at[j]], rows, sem).wait()
            pltpu.sync_copy(rows, acc_sh.at[dstv.at[j]], add=True)
            return carry

        lax.fori_loop(0, NCH, body, 0)
        plsc.subcore_barrier()
        pltpu.sync_copy(acc_sh.at[pl.ds(r0, PTR)],
                        out_acc.at[c].at[pl.ds(r0, PTR)])

    return seg_sum


@functools.lru_cache(maxsize=None)
def _build_seg_sum_deg():
    mesh = plsc.VectorSubcoreMesh(core_axis_name="c", subcore_axis_name="s",
                                  num_cores=NC, num_subcores=NS)

    @functools.partial(
        pl.kernel,
        out_type=[jax.ShapeDtypeStruct((NC, ACC_R, HID), jnp.float32),
                  jax.ShapeDtypeStruct((NC, ACC_R), jnp.float32)],
        mesh=mesh,
        scratch_types=[
            pltpu.VMEM((NCH, CH), jnp.int32),
            pltpu.VMEM((NCH, CH), jnp.int32),
            pltpu.VMEM((CH, HID), jnp.float32),
            pltpu.SemaphoreType.DMA,
            pltpu.VMEM_SHARED((ACC_R, HID), jnp.float32),
            pltpu.VMEM_SHARED((ACC_R,), jnp.float32),
            pltpu.VMEM((CH,), jnp.float32),
        ],
    )
    def seg_sum_deg(table, srci, dsti, z2d, z1d, out_acc, out_deg,
                    srcv, dstv, rows, sem, acc_sh, deg_sh, ones_v):
        c = lax.axis_index("c")
        s = lax.axis_index("s")
        wid = c * NS + s
        r0 = s * PTR
        pltpu.sync_copy(z2d, acc_sh.at[pl.ds(r0, PTR)])
        pltpu.sync_copy(z1d, deg_sh.at[pl.ds(r0, PTR)])
        pltpu.sync_copy(srci.at[wid], srcv)
        pltpu.sync_copy(dsti.at[wid], dstv)
        for i in range(CH // 16):
            ones_v[pl.ds(i * 16, 16)] = jnp.ones((16,), jnp.float32)
        plsc.subcore_barrier()

        def body(j, carry):
            pltpu.async_copy(table.at[srcv.at[j]], rows, sem).wait()
            pltpu.sync_copy(rows, acc_sh.at[dstv.at[j]], add=True)
            pltpu.sync_copy(ones_v, deg_sh.at[dstv.at[j]], add=True)
            return carry

        lax.fori_loop(0, NCH, body, 0)
        plsc.subcore_barrier()
        pltpu.sync_copy(acc_sh.at[pl.ds(r0, PTR)],
                        out_acc.at[c].at[pl.ds(r0, PTR)])
        pltpu.sync_copy(deg_sh.at[pl.ds(r0, PTR)],
                        out_deg.at[c].at[pl.ds(r0, PTR)])

    return seg_sum_deg


def kernel(x, he_index, he_attr, he_count, W1n, b1n, g1n, t1n, W2n, b2n,
           W1e, b1e, g1e, t1e, W2e, b2e, gf, bf):
    node_ids = he_index[0]
    he_ids = he_index[1]
    pad = MPAD - M
    pz = jnp.zeros((pad,), jnp.int32)
    pj = jnp.full((pad,), JUNK, jnp.int32)
    src1 = jnp.concatenate([node_ids, pz]).reshape(NW, NCH, CH)
    dst1 = jnp.concatenate([he_ids, pj]).reshape(NW, NCH, CH)
    src2 = jnp.concatenate([he_ids, pz]).reshape(NW, NCH, CH)
    dst2 = jnp.concatenate([node_ids, pj]).reshape(NW, NCH, CH)
    z2d = jnp.zeros((PTR, HID), jnp.float32)
    z1d = jnp.zeros((PTR,), jnp.float32)

    row = lambda v: v.reshape(1, -1)

    msg_n = _node_mlp(x, W1n, row(b1n), row(g1n), row(t1n), W2n, row(b2n))
    (agg_p,) = _build_seg_sum()(msg_n, src1, dst1, z2d)
    msg_e = _edge_mlp(agg_p, he_count.reshape(E, 1), he_attr,
                      W1e[:DHA], W1e[DHA:], row(b1e), row(g1e), row(t1e),
                      W2e, row(b2e))
    out_p, deg_p = _build_seg_sum_deg()(msg_e, src2, dst2, z2d, z1d)
    return _final(x, out_p, deg_p.reshape(NC, ACC_R, 1), gf.reshape(1, DN),
                  bf.reshape(1, DN))


# speedup vs baseline: 4.0596x; 1.0999x over previous
import functools

import jax
import jax.numpy as jnp
from jax import lax
from jax.experimental import pallas as pl
from jax.experimental.pallas import tpu as pltpu
from jax.experimental.pallas import tpu_sc as plsc

N = 10000
E = 10000
M = 320000
DN = 128
DHA = 16
HID = 128

NC = 2
NS = 16
NW = NC * NS
CH = 128
NCH = 80
MPAD = NW * NCH * CH
JUNK = 10000
ACC_R = 10240
PTR = ACC_R // NS
RB = 1000


def _ln(h, g, t, eps=1e-5):
    mu = jnp.mean(h, axis=-1, keepdims=True)
    var = jnp.mean((h - mu) ** 2, axis=-1, keepdims=True)
    return (h - mu) * lax.rsqrt(var + eps) * g + t


def _node_mlp_body(x_r, w1_r, b1_r, g1_r, t1_r, w2_r, b2_r, o_r):
    h = jnp.dot(x_r[...], w1_r[...], preferred_element_type=jnp.float32)
    h = _ln(h + b1_r[...], g1_r[...], t1_r[...])
    h = jnp.maximum(h, 0.0)
    o_r[...] = jnp.dot(h, w2_r[...], preferred_element_type=jnp.float32) + b2_r[...]


def _node_mlp(x, w1, b1, g1, t1, w2, b2):
    nb = N // RB
    full = lambda i: (0, 0)
    vec = lambda i: (0, 0)
    return pl.pallas_call(
        _node_mlp_body,
        grid=(nb,),
        in_specs=[
            pl.BlockSpec((RB, DN), lambda i: (i, 0)),
            pl.BlockSpec((DN, HID), full),
            pl.BlockSpec((1, HID), vec),
            pl.BlockSpec((1, HID), vec),
            pl.BlockSpec((1, HID), vec),
            pl.BlockSpec((HID, HID), full),
            pl.BlockSpec((1, HID), vec),
        ],
        out_specs=pl.BlockSpec((RB, HID), lambda i: (i, 0)),
        out_shape=jax.ShapeDtypeStruct((N, HID), jnp.float32),
    )(x, w1, b1, g1, t1, w2, b2)


def _edge_mlp_body(aggp_r, hc_r, attr_r, w1a_r, w1b_r, b1_r, g1_r, t1_r,
                   w2_r, b2_r, o_r):
    aggp = aggp_r[...]
    agg = (aggp[0] + aggp[1]) / (hc_r[...] + 1e-6)
    m = (jnp.dot(attr_r[...], w1a_r[...], preferred_element_type=jnp.float32)
         + jnp.dot(agg, w1b_r[...], preferred_element_type=jnp.float32))
    m = _ln(m + b1_r[...], g1_r[...], t1_r[...])
    m = jnp.maximum(m, 0.0)
    o = jnp.dot(m, w2_r[...], preferred_element_type=jnp.float32) + b2_r[...]
    o_r[...] = jnp.maximum(o, 0.0)


def _edge_mlp(agg_p, hc2, he_attr, w1a, w1b, b1, g1, t1, w2, b2):
    nb = E // RB
    full = lambda i: (0, 0)
    return pl.pallas_call(
        _edge_mlp_body,
        grid=(nb,),
        in_specs=[
            pl.BlockSpec((NC, RB, HID), lambda i: (0, i, 0)),
            pl.BlockSpec((RB, 1), lambda i: (i, 0)),
            pl.BlockSpec((RB, DHA), lambda i: (i, 0)),
            pl.BlockSpec((DHA, HID), full),
            pl.BlockSpec((HID, HID), full),
            pl.BlockSpec((1, HID), full),
            pl.BlockSpec((1, HID), full),
            pl.BlockSpec((1, HID), full),
            pl.BlockSpec((HID, DN), full),
            pl.BlockSpec((1, DN), full),
        ],
        out_specs=pl.BlockSpec((RB, DN), lambda i: (i, 0)),
        out_shape=jax.ShapeDtypeStruct((E, DN), jnp.float32),
    )(agg_p, hc2, he_attr, w1a, w1b, b1, g1, t1, w2, b2)


def _final_body(x_r, outp_r, degp_r, gf_r, bf_r, o_r):
    outp = outp_r[...]
    degp = degp_r[...]
    o = (outp[0] + outp[1]) / (degp[0] + degp[1] + 1e-6)
    o_r[...] = x_r[...] + _ln(o, gf_r[...], bf_r[...])


def _final(x, out_p, deg_p3, gf, bf):
    nb = N // RB
    return pl.pallas_call(
        _final_body,
        grid=(nb,),
        in_specs=[
            pl.BlockSpec((RB, DN), lambda i: (i, 0)),
            pl.BlockSpec((NC, RB, DN), lambda i: (0, i, 0)),
            pl.BlockSpec((NC, RB, 1), lambda i: (0, i, 0)),
            pl.BlockSpec((1, DN), lambda i: (0, 0)),
            pl.BlockSpec((1, DN), lambda i: (0, 0)),
        ],
        out_specs=pl.BlockSpec((RB, DN), lambda i: (i, 0)),
        out_shape=jax.ShapeDtypeStruct((N, DN), jnp.float32),
    )(x, out_p, deg_p3, gf, bf)


def _seg_pipeline(table, dsti_w, srcv, rowsA, rowsB, dA, dB,
                  gsemA, gsemB, dsemA, dsemB, ssem, scatter_chunk):
    def fire(j, rows, dbuf, gsem, dsem):
        jm = lax.rem(j, NCH)
        pltpu.async_copy(table.at[srcv.at[jm]], rows, gsem)
        pltpu.async_copy(dsti_w.at[jm], dbuf, dsem)

    def wait(rows, dbuf, gsem, dsem):
        pltpu.make_async_copy(table.at[pl.ds(0, CH)], rows, gsem).wait()
        pltpu.make_async_copy(dsti_w.at[0], dbuf, dsem).wait()

    fire(0, rowsA, dA, gsemA, dsemA)
    fire(1, rowsB, dB, gsemB, dsemB)

    def body(i, carry):
        j = 2 * i
        wait(rowsA, dA, gsemA, dsemA)
        for d in scatter_chunk(rowsA, dA, ssem):
            d.wait()
        fire(j + 2, rowsA, dA, gsemA, dsemA)
        wait(rowsB, dB, gsemB, dsemB)
        for d in scatter_chunk(rowsB, dB, ssem):
            d.wait()
        fire(j + 3, rowsB, dB, gsemB, dsemB)
        return carry

    lax.fori_loop(0, NCH // 2, body, 0)
    wait(rowsA, dA, gsemA, dsemA)
    wait(rowsB, dB, gsemB, dsemB)


@functools.lru_cache(maxsize=None)
def _build_seg_sum():
    mesh = plsc.VectorSubcoreMesh(core_axis_name="c", subcore_axis_name="s",
                                  num_cores=NC, num_subcores=NS)

    @functools.partial(
        pl.kernel,
        out_type=[jax.ShapeDtypeStruct((NC, ACC_R, HID), jnp.float32)],
        mesh=mesh,
        scratch_types=[
            pltpu.VMEM((NCH, CH), jnp.int32),
            pltpu.VMEM((CH, HID), jnp.float32),
            pltpu.VMEM((CH, HID), jnp.float32),
            pltpu.VMEM((CH,), jnp.int32),
            pltpu.VMEM((CH,), jnp.int32),
            pltpu.SemaphoreType.DMA,
            pltpu.SemaphoreType.DMA,
            pltpu.SemaphoreType.DMA,
            pltpu.SemaphoreType.DMA,
            pltpu.SemaphoreType.DMA,
            pltpu.VMEM_SHARED((ACC_R, HID), jnp.float32),
        ],
    )
    def seg_sum(table, srci, dsti, z2d, out_acc,
                srcv, rowsA, rowsB, dA, dB,
                gsemA, gsemB, dsemA, dsemB, ssem, acc_sh):
        c = lax.axis_index("c")
        s = lax.axis_index("s")
        wid = c * NS + s
        r0 = s * PTR
        pltpu.sync_copy(z2d, acc_sh.at[pl.ds(r0, PTR)])
        pltpu.sync_copy(srci.at[wid], srcv)
        plsc.subcore_barrier()

        def scatter_chunk(rows, dbuf, ssem_):
            return [pltpu.async_copy(rows, acc_sh.at[dbuf], ssem_, add=True)]

        _seg_pipeline(table, dsti.at[wid], srcv, rowsA, rowsB, dA, dB,
                      gsemA, gsemB, dsemA, dsemB, ssem, scatter_chunk)
        plsc.subcore_barrier()
        pltpu.sync_copy(acc_sh.at[pl.ds(r0, PTR)],
                        out_acc.at[c].at[pl.ds(r0, PTR)])

    return seg_sum


@functools.lru_cache(maxsize=None)
def _build_seg_sum_deg():
    mesh = plsc.VectorSubcoreMesh(core_axis_name="c", subcore_axis_name="s",
                                  num_cores=NC, num_subcores=NS)

    @functools.partial(
        pl.kernel,
        out_type=[jax.ShapeDtypeStruct((NC, ACC_R, HID), jnp.float32),
                  jax.ShapeDtypeStruct((NC, ACC_R), jnp.float32)],
        mesh=mesh,
        scratch_types=[
            pltpu.VMEM((NCH, CH), jnp.int32),
            pltpu.VMEM((CH, HID), jnp.float32),
            pltpu.VMEM((CH, HID), jnp.float32),
            pltpu.VMEM((CH,), jnp.int32),
            pltpu.VMEM((CH,), jnp.int32),
            pltpu.SemaphoreType.DMA,
            pltpu.SemaphoreType.DMA,
            pltpu.SemaphoreType.DMA,
            pltpu.SemaphoreType.DMA,
            pltpu.SemaphoreType.DMA,
            pltpu.VMEM_SHARED((ACC_R, HID), jnp.float32),
            pltpu.VMEM_SHARED((ACC_R,), jnp.float32),
            pltpu.VMEM((CH,), jnp.float32),
        ],
    )
    def seg_sum_deg(table, srci, dsti, z2d, z1d, out_acc, out_deg,
                    srcv, rowsA, rowsB, dA, dB,
                    gsemA, gsemB, dsemA, dsemB, ssem,
                    acc_sh, deg_sh, ones_v):
        c = lax.axis_index("c")
        s = lax.axis_index("s")
        wid = c * NS + s
        r0 = s * PTR
        pltpu.sync_copy(z2d, acc_sh.at[pl.ds(r0, PTR)])
        pltpu.sync_copy(z1d, deg_sh.at[pl.ds(r0, PTR)])
        pltpu.sync_copy(srci.at[wid], srcv)
        for i in range(CH // 16):
            ones_v[pl.ds(i * 16, 16)] = jnp.ones((16,), jnp.float32)
        plsc.subcore_barrier()

        def scatter_chunk(rows, dbuf, ssem_):
            d1 = pltpu.async_copy(rows, acc_sh.at[dbuf], ssem_, add=True)
            d2 = pltpu.async_copy(ones_v, deg_sh.at[dbuf], ssem_, add=True)
            return [d1, d2]

        _seg_pipeline(table, dsti.at[wid], srcv, rowsA, rowsB, dA, dB,
                      gsemA, gsemB, dsemA, dsemB, ssem, scatter_chunk)
        plsc.subcore_barrier()
        pltpu.sync_copy(acc_sh.at[pl.ds(r0, PTR)],
                        out_acc.at[c].at[pl.ds(r0, PTR)])
        pltpu.sync_copy(deg_sh.at[pl.ds(r0, PTR)],
                        out_deg.at[c].at[pl.ds(r0, PTR)])

    return seg_sum_deg


def kernel(x, he_index, he_attr, he_count, W1n, b1n, g1n, t1n, W2n, b2n,
           W1e, b1e, g1e, t1e, W2e, b2e, gf, bf):
    node_ids = he_index[0]
    he_ids = he_index[1]
    pad = MPAD - M
    pz = jnp.zeros((pad,), jnp.int32)
    pj = jnp.full((pad,), JUNK, jnp.int32)
    src1 = jnp.concatenate([node_ids, pz]).reshape(NW, NCH, CH)
    dst1 = jnp.concatenate([he_ids, pj]).reshape(NW, NCH, CH)
    src2 = jnp.concatenate([he_ids, pz]).reshape(NW, NCH, CH)
    dst2 = jnp.concatenate([node_ids, pj]).reshape(NW, NCH, CH)
    z2d = jnp.zeros((PTR, HID), jnp.float32)
    z1d = jnp.zeros((PTR,), jnp.float32)

    row = lambda v: v.reshape(1, -1)

    msg_n = _node_mlp(x, W1n, row(b1n), row(g1n), row(t1n), W2n, row(b2n))
    (agg_p,) = _build_seg_sum()(msg_n, src1, dst1, z2d)
    msg_e = _edge_mlp(agg_p, he_count.reshape(E, 1), he_attr,
                      W1e[:DHA], W1e[DHA:], row(b1e), row(g1e), row(t1e),
                      W2e, row(b2e))
    out_p, deg_p = _build_seg_sum_deg()(msg_e, src2, dst2, z2d, z1d)
    return _final(x, out_p, deg_p.reshape(NC, ACC_R, 1), gf.reshape(1, DN),
                  bf.reshape(1, DN))
